# baseline (speedup 1.0000x reference)
import functools

import jax
import jax.numpy as jnp
from jax import lax
from jax.experimental import pallas as pl
from jax.experimental.pallas import tpu as pltpu
from jax.experimental.pallas import tpu_sc as plsc

_IMAGE_TOKEN = 0
_IGNORE = -100
_B, _S, _D, _P = 4, 2048, 4096, 24
_E = _S + _P - 1
_EP = 2112
_SPAN = 264
_CHUNK = 8


def _index_body(ids_r, ids_l, lab_r, lab_l, msk_r, msk_l,
                tok, p_out, fmask, flab, fpos):
    idr = ids_r[...]
    j = lax.broadcasted_iota(jnp.int32, (_B, _EP), 1)
    is_img = idr == _IMAGE_TOKEN
    p = jnp.min(jnp.where(is_img, j, _EP), axis=1, keepdims=True)

    before = j < p
    in_img = jnp.logical_and(j >= p, j < p + _P)

    tok[...] = jnp.where(before, idr, jnp.where(in_img, 0, ids_l[...]))
    p_out[...] = jnp.broadcast_to(p, (_B, 32))

    fm = jnp.where(before, msk_r[...], jnp.where(in_img, 1, msk_l[...]))
    fl = jnp.where(before, lab_r[...], jnp.where(in_img, _IGNORE, lab_l[...]))

    r0 = lax.broadcasted_iota(jnp.int32, (_EP, _EP), 0)
    c0 = lax.broadcasted_iota(jnp.int32, (_EP, _EP), 1)
    lt = (r0 <= c0).astype(jnp.float32)
    cs = jnp.dot(fm.astype(jnp.float32), lt,
                 preferred_element_type=jnp.float32)
    pos = cs.astype(jnp.int32) - 1
    pos = jnp.where(fm == 0, 1, pos)

    fmask[...] = fm[:, :_E]
    flab[...] = fl[:, :_E]
    fpos[...] = pos[:, :_E]


def _build_indices(input_ids, attention_mask, labels):
    z = ((0, 0), (0, _EP - _S))
    zl = ((0, 0), (_P - 1, _EP - _S - _P + 1))
    ids_r = jnp.pad(input_ids, z, constant_values=1)
    ids_l = jnp.pad(input_ids, zl, constant_values=1)
    lab_r = jnp.pad(labels, z, constant_values=_IGNORE)
    lab_l = jnp.pad(labels, zl, constant_values=_IGNORE)
    msk_r = jnp.pad(attention_mask, z)
    msk_l = jnp.pad(attention_mask, zl)
    return pl.pallas_call(
        _index_body,
        out_shape=(
            jax.ShapeDtypeStruct((_B, _EP), jnp.int32),
            jax.ShapeDtypeStruct((_B, 32), jnp.int32),
            jax.ShapeDtypeStruct((_B, _E), jnp.int32),
            jax.ShapeDtypeStruct((_B, _E), jnp.int32),
            jax.ShapeDtypeStruct((_B, _E), jnp.int32),
        ),
    )(ids_r, ids_l, lab_r, lab_l, msk_r, msk_l)


def _pool_body(vis, out):
    v = vis[...]
    hsum = v[:, 0] + v[:, 1]
    for j in range(7):
        wp = (hsum[:, 2 * j] + hsum[:, 2 * j + 1]) * 0.25
        out[j] = wp.astype(jnp.bfloat16)


def _pool(visual_inputs):
    vis = visual_inputs.reshape(_B * _P, 14, 14, 2048)
    half = _B * _P // 2
    return pl.pallas_call(
        _pool_body,
        grid=(7, 2),
        in_specs=[pl.BlockSpec((half, 2, 14, 2048),
                               lambda k, m: (m, k, 0, 0))],
        out_specs=pl.BlockSpec((7, half, 2048), lambda k, m: (k, m, 0)),
        out_shape=jax.ShapeDtypeStruct((49, _B * _P, 2048), jnp.bfloat16),
    )(vis)


def _mm1_body(x, w1, out):
    k = pl.program_id(1)

    @pl.when(k == 0)
    def _():
        out[...] = jnp.zeros_like(out)

    out[...] += jnp.dot(x[0], w1[0].astype(jnp.bfloat16),
                        preferred_element_type=jnp.float32)


def _projector_mm1(pooled, W1):
    w1 = W1.reshape(49, 2048, 2048)
    return pl.pallas_call(
        _mm1_body,
        grid=(2, 49),
        in_specs=[
            pl.BlockSpec((1, _B * _P, 2048), lambda n, k: (k, 0, 0)),
            pl.BlockSpec((1, 2048, 1024), lambda n, k: (k, 0, n)),
        ],
        out_specs=pl.BlockSpec((_B * _P, 1024), lambda n, k: (0, n)),
        out_shape=jax.ShapeDtypeStruct((_B * _P, 2048), jnp.float32),
    )(pooled, w1)


def _mm2_body(a, b1, w2, b2, out):
    h = a[...] + b1[...]
    out[...] = jnp.dot(h.astype(jnp.bfloat16), w2[...].astype(jnp.bfloat16),
                       preferred_element_type=jnp.float32) + b2[...]


def _projector_mm2(A, b1, W2, b2):
    return pl.pallas_call(
        _mm2_body,
        out_shape=jax.ShapeDtypeStruct((_B * _P, _D), jnp.float32),
    )(A, b1.reshape(1, 2048), W2, b2.reshape(1, _D))


def _sc_merge_body(tok_hbm, p_hbm, emb_hbm, imgf_hbm, out_hbm,
                   idx_all_v, buf_v, pvec_v, sem):
    c = lax.axis_index("c")
    s = lax.axis_index("s")
    b = 2 * c + s // 8
    t = s % 8
    col0 = t * _SPAN
    base = _EP * b
    nfull = jnp.where(t == 7, 27, 33)

    pltpu.sync_copy(tok_hbm.at[pl.ds(base + col0, _SPAN)], idx_all_v)

    def gather_start(i):
        pltpu.async_copy(
            emb_hbm.at[idx_all_v.at[pl.ds(i * _CHUNK, _CHUNK)]],
            buf_v.at[i % 2], sem)

    def gather_wait(i):
        pltpu.make_async_copy(
            emb_hbm.at[idx_all_v.at[pl.ds(i * _CHUNK, _CHUNK)]],
            buf_v.at[i % 2], sem).wait()

    gather_start(0)

    def chunk(i, carry):
        gather_wait(i)

        @pl.when(i + 1 < nfull)
        def _():
            gather_start(i + 1)

        pltpu.sync_copy(buf_v.at[i % 2],
                        out_hbm.at[b, pl.ds(col0 + i * _CHUNK, _CHUNK)])
        return carry

    lax.fori_loop(0, nfull, chunk, 0)

    def text_chunk(chunk0):
        pltpu.sync_copy(tok_hbm.at[pl.ds(base + chunk0, _CHUNK)],
                        idx_all_v.at[pl.ds(0, _CHUNK)])
        pltpu.async_copy(emb_hbm.at[idx_all_v.at[pl.ds(0, _CHUNK)]],
                         buf_v.at[0], sem).wait()

    @pl.when(t == 7)
    def _():
        text_chunk(2064)
        pltpu.sync_copy(buf_v.at[0, pl.ds(0, 7)],
                        out_hbm.at[b, pl.ds(2064, 7)])

    pltpu.sync_copy(p_hbm.at[pl.ds(32 * b, 16)], pvec_v)
    p = pvec_v[...][0]
    span_hi = col0 + jnp.where(t == 7, 223, _SPAN)
    inter_lo = jnp.maximum(col0, p)
    inter_hi = jnp.minimum(span_hi, p + _P)

    def patch(chunk0, n):
        r_lo = jnp.maximum(chunk0, p)
        r_hi = jnp.minimum(chunk0 + n, p + _P)

        def row(r, carry):
            pltpu.sync_copy(imgf_hbm.at[b * _P + (r - p)],
                            buf_v.at[0, pl.ds(r - chunk0, 1)])
            return carry

        lax.fori_loop(r_lo, jnp.maximum(r_lo, r_hi), row, 0)

    @pl.when(inter_hi > inter_lo)
    def _():
        c_lo = (inter_lo // _CHUNK) * _CHUNK
        c_hi = jnp.minimum(((inter_hi + _CHUNK - 1) // _CHUNK) * _CHUNK, 2064)
        c_hi = jnp.maximum(c_lo, c_hi)

        def fix_chunk(m, carry):
            chunk0 = c_lo + m * _CHUNK
            text_chunk(chunk0)
            patch(chunk0, _CHUNK)
            pltpu.sync_copy(buf_v.at[0], out_hbm.at[b, pl.ds(chunk0, _CHUNK)])
            return carry

        lax.fori_loop(0, (c_hi - c_lo) // _CHUNK, fix_chunk, 0)

        @pl.when(inter_hi > 2064)
        def _():
            text_chunk(2064)
            patch(2064, 7)
            pltpu.sync_copy(buf_v.at[0, pl.ds(0, 7)],
                            out_hbm.at[b, pl.ds(2064, 7)])


def _sc_merge(tok, p_out, embed_table, image_features):
    mesh = plsc.VectorSubcoreMesh(core_axis_name="c", subcore_axis_name="s")
    f = pl.kernel(
        _sc_merge_body, mesh=mesh,
        out_type=jax.ShapeDtypeStruct((_B, _E, _D), jnp.float32),
        scratch_types=[
            pltpu.VMEM((_SPAN,), jnp.int32),
            pltpu.VMEM((2, _CHUNK, _D), jnp.float32),
            pltpu.VMEM((16,), jnp.int32),
            pltpu.SemaphoreType.DMA,
        ],
    )
    return f(tok.reshape(_B * _EP), p_out.reshape(_B * 32),
             embed_table, image_features.reshape(_B * _P, 1, _D))


def kernel(visual_inputs, input_ids, attention_mask, labels,
           embed_table, W1, b1, W2, b2):
    tok, p_out, fmask, flab, fpos = _build_indices(
        input_ids, attention_mask, labels)
    A = _projector_mm1(_pool(visual_inputs), W1)
    image_features = _projector_mm2(A, b1, W2, b2)
    final_embedding = _sc_merge(tok, p_out, embed_table, image_features)
    return (final_embedding, fmask, flab, fpos)


# speedup vs baseline: 1.0047x; 1.0047x over previous
import functools

import jax
import jax.numpy as jnp
from jax import lax
from jax.experimental import pallas as pl
from jax.experimental.pallas import tpu as pltpu
from jax.experimental.pallas import tpu_sc as plsc

_IMAGE_TOKEN = 0
_IGNORE = -100
_B, _S, _D, _P = 4, 2048, 4096, 24
_E = _S + _P - 1
_EP = 2112
_SPAN = 264
_CHUNK = 8


def _index_body(ids_r, ids_l, lab_r, lab_l, msk_r, msk_l,
                tok, p_out, fmask, flab, fpos):
    idr = ids_r[...]
    j = lax.broadcasted_iota(jnp.int32, (_B, _EP), 1)
    is_img = idr == _IMAGE_TOKEN
    p = jnp.min(jnp.where(is_img, j, _EP), axis=1, keepdims=True)

    before = j < p
    in_img = jnp.logical_and(j >= p, j < p + _P)

    tok[...] = jnp.where(before, idr, jnp.where(in_img, 0, ids_l[...]))
    p_out[...] = jnp.broadcast_to(p, (_B, 32))

    fm = jnp.where(before, msk_r[...], jnp.where(in_img, 1, msk_l[...]))
    fl = jnp.where(before, lab_r[...], jnp.where(in_img, _IGNORE, lab_l[...]))

    r0 = lax.broadcasted_iota(jnp.int32, (_EP, _EP), 0)
    c0 = lax.broadcasted_iota(jnp.int32, (_EP, _EP), 1)
    lt = (r0 <= c0).astype(jnp.float32)
    cs = jnp.dot(fm.astype(jnp.float32), lt,
                 preferred_element_type=jnp.float32)
    pos = cs.astype(jnp.int32) - 1
    pos = jnp.where(fm == 0, 1, pos)

    fmask[...] = fm[:, :_E]
    flab[...] = fl[:, :_E]
    fpos[...] = pos[:, :_E]


def _build_indices(input_ids, attention_mask, labels):
    z = ((0, 0), (0, _EP - _S))
    zl = ((0, 0), (_P - 1, _EP - _S - _P + 1))
    ids_r = jnp.pad(input_ids, z, constant_values=1)
    ids_l = jnp.pad(input_ids, zl, constant_values=1)
    lab_r = jnp.pad(labels, z, constant_values=_IGNORE)
    lab_l = jnp.pad(labels, zl, constant_values=_IGNORE)
    msk_r = jnp.pad(attention_mask, z)
    msk_l = jnp.pad(attention_mask, zl)
    return pl.pallas_call(
        _index_body,
        out_shape=(
            jax.ShapeDtypeStruct((_B, _EP), jnp.int32),
            jax.ShapeDtypeStruct((_B, 32), jnp.int32),
            jax.ShapeDtypeStruct((_B, _E), jnp.int32),
            jax.ShapeDtypeStruct((_B, _E), jnp.int32),
            jax.ShapeDtypeStruct((_B, _E), jnp.int32),
        ),
    )(ids_r, ids_l, lab_r, lab_l, msk_r, msk_l)


def _pool_body(vis, out):
    v = vis[...]
    hsum = v[:, 0] + v[:, 1]
    for j in range(7):
        wp = (hsum[:, 2 * j] + hsum[:, 2 * j + 1]) * 0.25
        out[j] = wp.astype(jnp.bfloat16)


def _pool(visual_inputs):
    vis = visual_inputs.reshape(_B * _P, 14, 14, 2048)
    half = _B * _P // 2
    return pl.pallas_call(
        _pool_body,
        grid=(7, 2),
        in_specs=[pl.BlockSpec((half, 2, 14, 2048),
                               lambda k, m: (m, k, 0, 0))],
        out_specs=pl.BlockSpec((7, half, 2048), lambda k, m: (k, m, 0)),
        out_shape=jax.ShapeDtypeStruct((49, _B * _P, 2048), jnp.bfloat16),
    )(vis)


def _mm1_body(x, w1, out):
    k = pl.program_id(0)

    @pl.when(k == 0)
    def _():
        out[...] = jnp.zeros_like(out)

    out[...] += jnp.dot(x[0], w1[0].astype(jnp.bfloat16),
                        preferred_element_type=jnp.float32)


def _projector_mm1(pooled, W1):
    w1 = W1.reshape(49, 2048, 2048)
    return pl.pallas_call(
        _mm1_body,
        grid=(49,),
        in_specs=[
            pl.BlockSpec((1, _B * _P, 2048), lambda k: (k, 0, 0)),
            pl.BlockSpec((1, 2048, 2048), lambda k: (k, 0, 0)),
        ],
        out_specs=pl.BlockSpec((_B * _P, 2048), lambda k: (0, 0)),
        out_shape=jax.ShapeDtypeStruct((_B * _P, 2048), jnp.float32),
    )(pooled, w1)


def _mm2_body(a, b1, w2, b2, out):
    h = a[...] + b1[...]
    out[...] = jnp.dot(h.astype(jnp.bfloat16), w2[...].astype(jnp.bfloat16),
                       preferred_element_type=jnp.float32) + b2[...]


def _projector_mm2(A, b1, W2, b2):
    return pl.pallas_call(
        _mm2_body,
        out_shape=jax.ShapeDtypeStruct((_B * _P, _D), jnp.float32),
    )(A, b1.reshape(1, 2048), W2, b2.reshape(1, _D))


def _sc_merge_body(tok_hbm, p_hbm, emb_hbm, imgf_hbm, out_hbm,
                   idx_all_v, buf_v, pvec_v, sem):
    c = lax.axis_index("c")
    s = lax.axis_index("s")
    b = 2 * c + s // 8
    t = s % 8
    col0 = t * _SPAN
    base = _EP * b
    nfull = jnp.where(t == 7, 27, 33)

    pltpu.sync_copy(tok_hbm.at[pl.ds(base + col0, _SPAN)], idx_all_v)

    def gather_start(i):
        pltpu.async_copy(
            emb_hbm.at[idx_all_v.at[pl.ds(i * _CHUNK, _CHUNK)]],
            buf_v.at[i % 2], sem)

    def gather_wait(i):
        pltpu.make_async_copy(
            emb_hbm.at[idx_all_v.at[pl.ds(i * _CHUNK, _CHUNK)]],
            buf_v.at[i % 2], sem).wait()

    gather_start(0)

    def chunk(i, carry):
        gather_wait(i)

        @pl.when(i + 1 < nfull)
        def _():
            gather_start(i + 1)

        pltpu.sync_copy(buf_v.at[i % 2],
                        out_hbm.at[b, pl.ds(col0 + i * _CHUNK, _CHUNK)])
        return carry

    lax.fori_loop(0, nfull, chunk, 0)

    def text_chunk(chunk0):
        pltpu.sync_copy(tok_hbm.at[pl.ds(base + chunk0, _CHUNK)],
                        idx_all_v.at[pl.ds(0, _CHUNK)])
        pltpu.async_copy(emb_hbm.at[idx_all_v.at[pl.ds(0, _CHUNK)]],
                         buf_v.at[0], sem).wait()

    @pl.when(t == 7)
    def _():
        text_chunk(2064)
        pltpu.sync_copy(buf_v.at[0, pl.ds(0, 7)],
                        out_hbm.at[b, pl.ds(2064, 7)])

    pltpu.sync_copy(p_hbm.at[pl.ds(32 * b, 16)], pvec_v)
    p = pvec_v[...][0]
    span_hi = col0 + jnp.where(t == 7, 223, _SPAN)
    inter_lo = jnp.maximum(col0, p)
    inter_hi = jnp.minimum(span_hi, p + _P)

    def patch(chunk0, n):
        r_lo = jnp.maximum(chunk0, p)
        r_hi = jnp.minimum(chunk0 + n, p + _P)

        def row(r, carry):
            pltpu.sync_copy(imgf_hbm.at[b * _P + (r - p)],
                            buf_v.at[0, pl.ds(r - chunk0, 1)])
            return carry

        lax.fori_loop(r_lo, jnp.maximum(r_lo, r_hi), row, 0)

    @pl.when(inter_hi > inter_lo)
    def _():
        c_lo = (inter_lo // _CHUNK) * _CHUNK
        c_hi = jnp.minimum(((inter_hi + _CHUNK - 1) // _CHUNK) * _CHUNK, 2064)
        c_hi = jnp.maximum(c_lo, c_hi)

        def fix_chunk(m, carry):
            chunk0 = c_lo + m * _CHUNK
            text_chunk(chunk0)
            patch(chunk0, _CHUNK)
            pltpu.sync_copy(buf_v.at[0], out_hbm.at[b, pl.ds(chunk0, _CHUNK)])
            return carry

        lax.fori_loop(0, (c_hi - c_lo) // _CHUNK, fix_chunk, 0)

        @pl.when(inter_hi > 2064)
        def _():
            text_chunk(2064)
            patch(2064, 7)
            pltpu.sync_copy(buf_v.at[0, pl.ds(0, 7)],
                            out_hbm.at[b, pl.ds(2064, 7)])


def _sc_merge(tok, p_out, embed_table, image_features):
    mesh = plsc.VectorSubcoreMesh(core_axis_name="c", subcore_axis_name="s")
    f = pl.kernel(
        _sc_merge_body, mesh=mesh,
        out_type=jax.ShapeDtypeStruct((_B, _E, _D), jnp.float32),
        scratch_types=[
            pltpu.VMEM((_SPAN,), jnp.int32),
            pltpu.VMEM((2, _CHUNK, _D), jnp.float32),
            pltpu.VMEM((16,), jnp.int32),
            pltpu.SemaphoreType.DMA,
        ],
    )
    return f(tok.reshape(_B * _EP), p_out.reshape(_B * 32),
             embed_table, image_features.reshape(_B * _P, 1, _D))


def kernel(visual_inputs, input_ids, attention_mask, labels,
           embed_table, W1, b1, W2, b2):
    tok, p_out, fmask, flab, fpos = _build_indices(
        input_ids, attention_mask, labels)
    A = _projector_mm1(_pool(visual_inputs), W1)
    image_features = _projector_mm2(A, b1, W2, b2)
    final_embedding = _sc_merge(tok, p_out, embed_table, image_features)
    return (final_embedding, fmask, flab, fpos)


# speedup vs baseline: 1.0058x; 1.0010x over previous
import jax
import jax.numpy as jnp
from jax import lax
from jax.experimental import pallas as pl
from jax.experimental.pallas import tpu as pltpu
from jax.experimental.pallas import tpu_sc as plsc

_IMAGE_TOKEN = 0
_IGNORE = -100
_B, _S, _D, _P = 4, 2048, 4096, 24
_E = _S + _P - 1
_EP = 2112
_SPAN = 264
_CHUNK = 8


def _index_body(ids_r, ids_l, lab_r, lab_l, msk_r, msk_l,
                tok, p_out, fmask, flab, fpos):
    idr = ids_r[...]
    j = lax.broadcasted_iota(jnp.int32, (_B, _EP), 1)
    is_img = idr == _IMAGE_TOKEN
    p = jnp.min(jnp.where(is_img, j, _EP), axis=1, keepdims=True)

    before = j < p
    in_img = jnp.logical_and(j >= p, j < p + _P)

    tok[...] = jnp.where(before, idr, jnp.where(in_img, 0, ids_l[...]))
    p_out[...] = jnp.broadcast_to(p, (_B, 32))

    fm = jnp.where(before, msk_r[...], jnp.where(in_img, 1, msk_l[...]))
    fl = jnp.where(before, lab_r[...], jnp.where(in_img, _IGNORE, lab_l[...]))

    r0 = lax.broadcasted_iota(jnp.int32, (_EP, _EP), 0)
    c0 = lax.broadcasted_iota(jnp.int32, (_EP, _EP), 1)
    lt = (r0 <= c0).astype(jnp.float32)
    cs = jnp.dot(fm.astype(jnp.float32), lt,
                 preferred_element_type=jnp.float32)
    pos = cs.astype(jnp.int32) - 1
    pos = jnp.where(fm == 0, 1, pos)

    fmask[...] = fm[:, :_E]
    flab[...] = fl[:, :_E]
    fpos[...] = pos[:, :_E]


def _build_indices(input_ids, attention_mask, labels):
    z = ((0, 0), (0, _EP - _S))
    zl = ((0, 0), (_P - 1, _EP - _S - _P + 1))
    ids_r = jnp.pad(input_ids, z, constant_values=1)
    ids_l = jnp.pad(input_ids, zl, constant_values=1)
    lab_r = jnp.pad(labels, z, constant_values=_IGNORE)
    lab_l = jnp.pad(labels, zl, constant_values=_IGNORE)
    msk_r = jnp.pad(attention_mask, z)
    msk_l = jnp.pad(attention_mask, zl)
    return pl.pallas_call(
        _index_body,
        out_shape=(
            jax.ShapeDtypeStruct((_B, _EP), jnp.int32),
            jax.ShapeDtypeStruct((_B, 32), jnp.int32),
            jax.ShapeDtypeStruct((_B, _E), jnp.int32),
            jax.ShapeDtypeStruct((_B, _E), jnp.int32),
            jax.ShapeDtypeStruct((_B, _E), jnp.int32),
        ),
    )(ids_r, ids_l, lab_r, lab_l, msk_r, msk_l)


def _pool_body(vis, out):
    v = vis[...]
    hsum = v[:, 0] + v[:, 1]
    for j in range(7):
        wp = (hsum[:, 2 * j] + hsum[:, 2 * j + 1]) * 0.25
        out[j] = wp.astype(jnp.bfloat16)


def _pool(visual_inputs):
    vis = visual_inputs.reshape(_B * _P, 14, 14, 2048)
    half = _B * _P // 2
    return pl.pallas_call(
        _pool_body,
        grid=(7, 2),
        in_specs=[pl.BlockSpec((half, 2, 14, 2048),
                               lambda k, m: (m, k, 0, 0))],
        out_specs=pl.BlockSpec((7, half, 2048), lambda k, m: (k, m, 0)),
        out_shape=jax.ShapeDtypeStruct((49, _B * _P, 2048), jnp.bfloat16),
    )(vis)


def _mm1_body(x, w1, out):
    k = pl.program_id(0)

    @pl.when(k == 0)
    def _():
        out[...] = jnp.zeros_like(out)

    out[...] += jnp.dot(x[0], w1[0].astype(jnp.bfloat16),
                        preferred_element_type=jnp.float32)


def _projector_mm1(pooled, W1):
    w1 = W1.reshape(49, 2048, 2048)
    return pl.pallas_call(
        _mm1_body,
        grid=(49,),
        in_specs=[
            pl.BlockSpec((1, _B * _P, 2048), lambda k: (k, 0, 0)),
            pl.BlockSpec((1, 2048, 2048), lambda k: (k, 0, 0)),
        ],
        out_specs=pl.BlockSpec((_B * _P, 2048), lambda k: (0, 0)),
        out_shape=jax.ShapeDtypeStruct((_B * _P, 2048), jnp.float32),
    )(pooled, w1)


def _mm2_body(a, b1, w2, b2, out):
    h = a[...] + b1[...]
    out[...] = jnp.dot(h.astype(jnp.bfloat16), w2[...].astype(jnp.bfloat16),
                       preferred_element_type=jnp.float32) + b2[...]


def _projector_mm2(A, b1, W2, b2):
    return pl.pallas_call(
        _mm2_body,
        out_shape=jax.ShapeDtypeStruct((_B * _P, _D), jnp.float32),
    )(A, b1.reshape(1, 2048), W2, b2.reshape(1, _D))


def _sc_merge_body(tok_hbm, p_hbm, emb_hbm, imgf_hbm, out_hbm,
                   idx_all_v, buf_v, pvec_v, sem):
    c = lax.axis_index("c")
    s = lax.axis_index("s")
    b = 2 * c + s // 8
    t = s % 8
    col0 = t * _SPAN
    base = _EP * b
    nfull = jnp.where(t == 7, 27, 33)

    pltpu.sync_copy(tok_hbm.at[pl.ds(base + col0, _SPAN)], idx_all_v)

    def gather_start(i):
        pltpu.async_copy(
            emb_hbm.at[idx_all_v.at[pl.ds(i * _CHUNK, _CHUNK)]],
            buf_v.at[i % 2], sem)

    def gather_wait(i):
        pltpu.make_async_copy(
            emb_hbm.at[idx_all_v.at[pl.ds(i * _CHUNK, _CHUNK)]],
            buf_v.at[i % 2], sem).wait()

    gather_start(0)

    def chunk(i, carry):
        gather_wait(i)

        @pl.when(i + 1 < nfull)
        def _():
            gather_start(i + 1)

        pltpu.sync_copy(buf_v.at[i % 2],
                        out_hbm.at[b, pl.ds(col0 + i * _CHUNK, _CHUNK)])
        return carry

    lax.fori_loop(0, nfull, chunk, 0)

    def text_chunk(chunk0):
        pltpu.sync_copy(tok_hbm.at[pl.ds(base + chunk0, _CHUNK)],
                        idx_all_v.at[pl.ds(0, _CHUNK)])
        pltpu.async_copy(emb_hbm.at[idx_all_v.at[pl.ds(0, _CHUNK)]],
                         buf_v.at[0], sem).wait()

    @pl.when(t == 7)
    def _():
        text_chunk(2064)
        pltpu.sync_copy(buf_v.at[0, pl.ds(0, 7)],
                        out_hbm.at[b, pl.ds(2064, 7)])

    pltpu.sync_copy(p_hbm.at[pl.ds(32 * b, 16)], pvec_v)
    p = pvec_v[...][0]
    span_hi = col0 + jnp.where(t == 7, 223, _SPAN)
    inter_lo = jnp.maximum(col0, p)
    inter_hi = jnp.minimum(span_hi, p + _P)

    def patch(chunk0, n):
        r_lo = jnp.maximum(chunk0, p)
        r_hi = jnp.minimum(chunk0 + n, p + _P)

        def row(r, carry):
            pltpu.sync_copy(imgf_hbm.at[b * _P + (r - p)],
                            buf_v.at[0, pl.ds(r - chunk0, 1)])
            return carry

        lax.fori_loop(r_lo, jnp.maximum(r_lo, r_hi), row, 0)

    @pl.when(inter_hi > inter_lo)
    def _():
        c_lo = (inter_lo // _CHUNK) * _CHUNK
        c_hi = jnp.minimum(((inter_hi + _CHUNK - 1) // _CHUNK) * _CHUNK, 2064)
        c_hi = jnp.maximum(c_lo, c_hi)

        def fix_chunk(m, carry):
            chunk0 = c_lo + m * _CHUNK
            text_chunk(chunk0)
            patch(chunk0, _CHUNK)
            pltpu.sync_copy(buf_v.at[0], out_hbm.at[b, pl.ds(chunk0, _CHUNK)])
            return carry

        lax.fori_loop(0, (c_hi - c_lo) // _CHUNK, fix_chunk, 0)

        @pl.when(inter_hi > 2064)
        def _():
            text_chunk(2064)
            patch(2064, 7)
            pltpu.sync_copy(buf_v.at[0, pl.ds(0, 7)],
                            out_hbm.at[b, pl.ds(2064, 7)])


def _sc_merge(tok, p_out, embed_table, image_features):
    mesh = plsc.VectorSubcoreMesh(core_axis_name="c", subcore_axis_name="s")
    f = pl.kernel(
        _sc_merge_body, mesh=mesh,
        out_type=jax.ShapeDtypeStruct((_B, _E, _D), jnp.float32),
        scratch_types=[
            pltpu.VMEM((_SPAN,), jnp.int32),
            pltpu.VMEM((2, _CHUNK, _D), jnp.float32),
            pltpu.VMEM((16,), jnp.int32),
            pltpu.SemaphoreType.DMA,
        ],
    )
    return f(tok.reshape(_B * _EP), p_out.reshape(_B * 32),
             embed_table, image_features.reshape(_B * _P, 1, _D))


def kernel(visual_inputs, input_ids, attention_mask, labels,
           embed_table, W1, b1, W2, b2):
    tok, p_out, fmask, flab, fpos = _build_indices(
        input_ids, attention_mask, labels)
    A = _projector_mm1(_pool(visual_inputs), W1)
    image_features = _projector_mm2(A, b1, W2, b2)
    final_embedding = _sc_merge(tok, p_out, embed_table, image_features)
    return (final_embedding, fmask, flab, fpos)
